# Initial kernel scaffold; baseline (speedup 1.0000x reference)
import jax
import jax.numpy as jnp
from jax.experimental import pallas as pl


def kernel(x, edge_index, edge_attr, W_down, b_down, W_up, b_up, W_time, b_time, W_q, b_q, W_k, b_k, W_v, b_v, cluster_emb, W_out, b_out):
    raise NotImplementedError("write your pallas kernel here")


# speedup vs baseline: 19.8320x; 19.8320x over previous
import functools

import jax
import jax.numpy as jnp
from jax import lax
from jax.experimental import pallas as pl
from jax.experimental.pallas import tpu as pltpu
from jax.experimental.pallas import tpu_sc as plsc

N = 10000
E = 320000
IN_CH = 128
AD = 64
NCL = 8
SCAL = float(AD) ** (-0.5)

NWORK = 32
EPT = E // NWORK
NB = 400
EB = 512

_mesh = plsc.VectorSubcoreMesh(core_axis_name="c", subcore_axis_name="s")


def _q_body(x_ref, wd_ref, bd_ref, wq_ref, bq_ref, q_ref):
    nf = jnp.dot(x_ref[...], wd_ref[...], preferred_element_type=jnp.float32)
    nf = jnp.maximum(nf + bd_ref[...], 0.0)
    q_ref[...] = jnp.dot(nf, wq_ref[...], preferred_element_type=jnp.float32) + bq_ref[...]


def _q_call(x, wd, bd, wq, bq):
    return pl.pallas_call(
        _q_body,
        grid=(N // NB,),
        in_specs=[
            pl.BlockSpec((NB, IN_CH), lambda i: (i, 0)),
            pl.BlockSpec((IN_CH, AD), lambda i: (0, 0)),
            pl.BlockSpec((1, AD), lambda i: (0, 0)),
            pl.BlockSpec((AD, AD), lambda i: (0, 0)),
            pl.BlockSpec((1, AD), lambda i: (0, 0)),
        ],
        out_specs=pl.BlockSpec((NB, AD), lambda i: (i, 0)),
        out_shape=jax.ShapeDtypeStruct((N, AD), jnp.float32),
    )(x, wd, bd, wq, bq)


def _edge_body(ea_ref, qi_ref, wt_ref, bt_ref, wk_ref, bk_ref, wv_ref,
               bv_ref, cem_ref, v_ref, e_ref, a_ref, r16_ref, cnt_ref):
    i = pl.program_id(0)
    tf = jnp.dot(ea_ref[...], wt_ref[...], preferred_element_type=jnp.float32)
    tf = jnp.maximum(tf + bt_ref[...], 0.0)
    k = jnp.dot(tf, wk_ref[...], preferred_element_type=jnp.float32) + bk_ref[...]
    v_ref[...] = jnp.dot(tf, wv_ref[...], preferred_element_type=jnp.float32) + bv_ref[...]
    sim = jnp.dot(tf, cem_ref[...], preferred_element_type=jnp.float32)
    best = sim[:, 0]
    bidx = jnp.zeros((EB,), jnp.int32)
    for c in range(1, NCL):
        m = sim[:, c] > best
        best = jnp.where(m, sim[:, c], best)
        bidx = jnp.where(m, jnp.int32(c), bidx)
    a_ref[...] = bidx
    att = jnp.sum(qi_ref[...] * k, axis=1) * SCAL
    ee = jnp.exp(att)
    e_ref[...] = ee
    col = lax.broadcasted_iota(jnp.int32, (EB, 16), 1)
    a2 = bidx[:, None]
    r16_ref[...] = (jnp.where(col == a2, ee[:, None], 0.0)
                    + jnp.where(col == a2 + NCL, 1.0, 0.0))
    onehot = (lax.broadcasted_iota(jnp.int32, (EB, NCL), 1) == a2)
    part = jnp.sum(onehot.astype(jnp.float32), axis=0, keepdims=True)

    @pl.when(i == 0)
    def _():
        cnt_ref[...] = jnp.zeros_like(cnt_ref)

    cnt_ref[...] += part


def _edge_call(ea, qi, wt, bt, wk, bk, wv, bv, cem):
    return pl.pallas_call(
        _edge_body,
        grid=(E // EB,),
        in_specs=[
            pl.BlockSpec((EB, 16), lambda i: (i, 0)),
            pl.BlockSpec((EB, AD), lambda i: (i, 0)),
            pl.BlockSpec((16, AD), lambda i: (0, 0)),
            pl.BlockSpec((1, AD), lambda i: (0, 0)),
            pl.BlockSpec((AD, AD), lambda i: (0, 0)),
            pl.BlockSpec((1, AD), lambda i: (0, 0)),
            pl.BlockSpec((AD, AD), lambda i: (0, 0)),
            pl.BlockSpec((1, AD), lambda i: (0, 0)),
            pl.BlockSpec((AD, NCL), lambda i: (0, 0)),
        ],
        out_specs=[
            pl.BlockSpec((EB, AD), lambda i: (i, 0)),
            pl.BlockSpec((EB,), lambda i: (i,)),
            pl.BlockSpec((EB,), lambda i: (i,)),
            pl.BlockSpec((EB, 16), lambda i: (i, 0)),
            pl.BlockSpec((1, NCL), lambda i: (0, 0)),
        ],
        out_shape=[
            jax.ShapeDtypeStruct((E, AD), jnp.float32),
            jax.ShapeDtypeStruct((E,), jnp.float32),
            jax.ShapeDtypeStruct((E,), jnp.int32),
            jax.ShapeDtypeStruct((E, 16), jnp.float32),
            jax.ShapeDtypeStruct((1, NCL), jnp.float32),
        ],
    )(ea, qi, wt, bt, wk, bk, wv, bv, cem)


_GQ_CH = 1000


@functools.partial(
    pl.kernel,
    out_type=jax.ShapeDtypeStruct((E, AD), jnp.float32),
    mesh=_mesh,
    compiler_params=pltpu.CompilerParams(use_tc_tiling_on_sc=False, needs_layout_passes=False),
    scratch_types=[
        pltpu.VMEM((_GQ_CH,), jnp.int32),
        pltpu.VMEM((_GQ_CH, AD), jnp.float32),
        pltpu.SemaphoreType.DMA,
    ],
)
def _gather_q(q_hbm, src_hbm, out_hbm, idx_v, rows_v, sem):
    wid = lax.axis_index("s") * 2 + lax.axis_index("c")
    base = wid * EPT

    def body(i, carry):
        off = pl.multiple_of(base + i * _GQ_CH, 8)
        pltpu.sync_copy(src_hbm.at[pl.ds(off, _GQ_CH)], idx_v)
        pltpu.async_copy(q_hbm.at[idx_v], rows_v, sem).wait()
        pltpu.sync_copy(rows_v, out_hbm.at[pl.ds(off, _GQ_CH)])
        return carry

    lax.fori_loop(0, EPT // _GQ_CH, body, 0)


_H_CH = 2000


@functools.partial(
    pl.kernel,
    out_type=jax.ShapeDtypeStruct((2, N, 16), jnp.float32),
    mesh=_mesh,
    compiler_params=pltpu.CompilerParams(use_tc_tiling_on_sc=False, needs_layout_passes=False),
    scratch_types=[
        pltpu.VMEM((_H_CH, 16), jnp.float32),
        pltpu.VMEM((_H_CH,), jnp.int32),
        pltpu.VMEM_SHARED((N, 16), jnp.float32),
    ],
)
def _hist(r16_hbm, src_hbm, z_hbm, out_hbm, rows_v, idx_v, bins_sh):
    cid = lax.axis_index("c")
    sid = lax.axis_index("s")
    wid = sid * 2 + cid

    @pl.when(sid == 0)
    def _():
        pltpu.sync_copy(z_hbm, bins_sh)

    plsc.subcore_barrier()

    def body(i, carry):
        off = pl.multiple_of(wid * EPT + i * _H_CH, 8)
        pltpu.sync_copy(src_hbm.at[pl.ds(off, _H_CH)], idx_v)
        pltpu.sync_copy(r16_hbm.at[pl.ds(off, _H_CH)], rows_v)
        pltpu.sync_copy(rows_v, bins_sh.at[idx_v], add=True)
        return carry

    lax.fori_loop(0, EPT // _H_CH, body, 0)
    plsc.subcore_barrier()
    rpt = N // 16
    ro = pl.multiple_of(sid * rpt, 8)
    pltpu.sync_copy(bins_sh.at[pl.ds(ro, rpt)], out_hbm.at[cid].at[pl.ds(ro, rpt)])


_C_CH = 2000


@functools.partial(
    pl.kernel,
    out_type=jax.ShapeDtypeStruct((E,), jnp.float32),
    mesh=_mesh,
    compiler_params=pltpu.CompilerParams(use_tc_tiling_on_sc=False, needs_layout_passes=False),
    scratch_types=[
        pltpu.VMEM((_C_CH, 16), jnp.float32),
        pltpu.VMEM((_C_CH, 16), jnp.float32),
        pltpu.VMEM((_C_CH,), jnp.int32),
        pltpu.VMEM((_C_CH,), jnp.int32),
        pltpu.VMEM((_C_CH,), jnp.float32),
        pltpu.VMEM((_C_CH,), jnp.float32),
        pltpu.SemaphoreType.DMA,
    ],
)
def _coef(b0_hbm, b1_hbm, src_hbm, asg_hbm, e_hbm, out_hbm,
          r0_v, r1_v, src_v, asg_v, e_v, out_v, sem):
    wid = lax.axis_index("s") * 2 + lax.axis_index("c")

    def body(i, carry):
        off = pl.multiple_of(wid * EPT + i * _C_CH, 8)
        pltpu.sync_copy(src_hbm.at[pl.ds(off, _C_CH)], src_v)
        pltpu.sync_copy(asg_hbm.at[pl.ds(off, _C_CH)], asg_v)
        pltpu.sync_copy(e_hbm.at[pl.ds(off, _C_CH)], e_v)
        pltpu.async_copy(b0_hbm.at[src_v], r0_v, sem).wait()
        pltpu.async_copy(b1_hbm.at[src_v], r1_v, sem).wait()

        def grp(g, c2):
            gb = pl.multiple_of(g * 16, 8)
            rows = lax.broadcasted_iota(jnp.int32, (16,), 0) + gb
            a16 = asg_v[pl.ds(gb, 16)]
            den = (plsc.load_gather(r0_v, [rows, a16])
                   + plsc.load_gather(r1_v, [rows, a16]))
            cnt = (plsc.load_gather(r0_v, [rows, a16 + NCL])
                   + plsc.load_gather(r1_v, [rows, a16 + NCL]))
            den = jnp.where(den > 0.0, den, 1.0)
            cnt = jnp.maximum(cnt, 1.0)
            e16 = e_v[pl.ds(gb, 16)]
            out_v[pl.ds(gb, 16)] = e16 / den / cnt
            return c2

        lax.fori_loop(0, _C_CH // 16, grp, 0)
        pltpu.sync_copy(out_v, out_hbm.at[pl.ds(off, _C_CH)])
        return carry

    lax.fori_loop(0, EPT // _C_CH, body, 0)


def _scale_body(v_ref, c_ref, wv_ref):
    wv_ref[...] = v_ref[...] * c_ref[...][:, None]


def _scale_call(v, coef):
    return pl.pallas_call(
        _scale_body,
        grid=(E // EB,),
        in_specs=[
            pl.BlockSpec((EB, AD), lambda i: (i, 0)),
            pl.BlockSpec((EB,), lambda i: (i,)),
        ],
        out_specs=pl.BlockSpec((EB, AD), lambda i: (i, 0)),
        out_shape=jax.ShapeDtypeStruct((E, AD), jnp.float32),
    )(v, coef)


_S_CH = 1000


@functools.partial(
    pl.kernel,
    out_type=jax.ShapeDtypeStruct((2, N, AD), jnp.float32),
    mesh=_mesh,
    compiler_params=pltpu.CompilerParams(use_tc_tiling_on_sc=False, needs_layout_passes=False),
    scratch_types=[
        pltpu.VMEM((_S_CH, AD), jnp.float32),
        pltpu.VMEM((_S_CH,), jnp.int32),
        pltpu.VMEM_SHARED((N, AD), jnp.float32),
    ],
)
def _scatter_v(wv_hbm, src_hbm, z_hbm, out_hbm, rows_v, idx_v, comb_sh):
    cid = lax.axis_index("c")
    sid = lax.axis_index("s")
    wid = sid * 2 + cid

    @pl.when(sid == 0)
    def _():
        pltpu.sync_copy(z_hbm, comb_sh)

    plsc.subcore_barrier()

    def body(i, carry):
        off = pl.multiple_of(wid * EPT + i * _S_CH, 8)
        pltpu.sync_copy(src_hbm.at[pl.ds(off, _S_CH)], idx_v)
        pltpu.sync_copy(wv_hbm.at[pl.ds(off, _S_CH)], rows_v)
        pltpu.sync_copy(rows_v, comb_sh.at[idx_v], add=True)
        return carry

    lax.fori_loop(0, EPT // _S_CH, body, 0)
    plsc.subcore_barrier()
    rpt = N // 16
    ro = pl.multiple_of(sid * rpt, 8)
    pltpu.sync_copy(comb_sh.at[pl.ds(ro, rpt)], out_hbm.at[cid].at[pl.ds(ro, rpt)])


def _final_body(x_ref, c0_ref, c1_ref, cnt_ref, wo_ref, bo_ref, wu_ref,
                bu_ref, out_ref):
    ne = jnp.sum((cnt_ref[...] > 0.0).astype(jnp.float32))
    comb = (c0_ref[...] + c1_ref[...]) / jnp.maximum(ne, 1.0)
    fused = jnp.dot(comb, wo_ref[...], preferred_element_type=jnp.float32)
    fused = jnp.maximum(fused + bo_ref[...], 0.0)
    out = jnp.dot(fused, wu_ref[...], preferred_element_type=jnp.float32)
    out_ref[...] = x_ref[...] + out + bu_ref[...]


def _final_call(x, c0, c1, cnt8, wo, bo, wu, bu):
    return pl.pallas_call(
        _final_body,
        grid=(N // NB,),
        in_specs=[
            pl.BlockSpec((NB, IN_CH), lambda i: (i, 0)),
            pl.BlockSpec((NB, AD), lambda i: (i, 0)),
            pl.BlockSpec((NB, AD), lambda i: (i, 0)),
            pl.BlockSpec((1, NCL), lambda i: (0, 0)),
            pl.BlockSpec((AD, AD), lambda i: (0, 0)),
            pl.BlockSpec((1, AD), lambda i: (0, 0)),
            pl.BlockSpec((AD, IN_CH), lambda i: (0, 0)),
            pl.BlockSpec((1, IN_CH), lambda i: (0, 0)),
        ],
        out_specs=pl.BlockSpec((NB, IN_CH), lambda i: (i, 0)),
        out_shape=jax.ShapeDtypeStruct((N, IN_CH), jnp.float32),
    )(x, c0, c1, cnt8, wo, bo, wu, bu)


@jax.jit
def kernel(x, edge_index, edge_attr, W_down, b_down, W_up, b_up, W_time,
           b_time, W_q, b_q, W_k, b_k, W_v, b_v, cluster_emb, W_out, b_out):
    src = edge_index[0].astype(jnp.int32)
    q = _q_call(x, W_down, b_down.reshape(1, AD), W_q, b_q.reshape(1, AD))
    qi = _gather_q(q, src)
    v, e, assign, r16, cnt8 = _edge_call(
        edge_attr, qi, W_time, b_time.reshape(1, AD), W_k, b_k.reshape(1, AD),
        W_v, b_v.reshape(1, AD), cluster_emb.T)
    bins = _hist(r16, src, jnp.zeros((N, 16), jnp.float32))
    coef = _coef(bins[0], bins[1], src, assign, e)
    wv = _scale_call(v, coef)
    comb = _scatter_v(wv, src, jnp.zeros((N, AD), jnp.float32))
    return _final_call(x, comb[0], comb[1], cnt8, W_out,
                       b_out.reshape(1, AD), W_up, b_up.reshape(1, IN_CH))


# speedup vs baseline: 22.0674x; 1.1127x over previous
import functools

import jax
import jax.numpy as jnp
from jax import lax
from jax.experimental import pallas as pl
from jax.experimental.pallas import tpu as pltpu
from jax.experimental.pallas import tpu_sc as plsc

N = 10000
E = 320000
IN_CH = 128
AD = 64
NCL = 8
SCAL = float(AD) ** (-0.5)

NWORK = 32
EPT = E // NWORK
NB = 1000
EB = 512

_mesh = plsc.VectorSubcoreMesh(core_axis_name="c", subcore_axis_name="s")


def _q_body(x_ref, wd_ref, bd_ref, wq_ref, bq_ref, q_ref):
    nf = jnp.dot(x_ref[...], wd_ref[...], preferred_element_type=jnp.float32)
    nf = jnp.maximum(nf + bd_ref[...], 0.0)
    q_ref[...] = jnp.dot(nf, wq_ref[...], preferred_element_type=jnp.float32) + bq_ref[...]


def _q_call(x, wd, bd, wq, bq):
    return pl.pallas_call(
        _q_body,
        grid=(N // NB,),
        in_specs=[
            pl.BlockSpec((NB, IN_CH), lambda i: (i, 0)),
            pl.BlockSpec((IN_CH, AD), lambda i: (0, 0)),
            pl.BlockSpec((1, AD), lambda i: (0, 0)),
            pl.BlockSpec((AD, AD), lambda i: (0, 0)),
            pl.BlockSpec((1, AD), lambda i: (0, 0)),
        ],
        out_specs=pl.BlockSpec((NB, AD), lambda i: (i, 0)),
        out_shape=jax.ShapeDtypeStruct((N, AD), jnp.float32),
    )(x, wd, bd, wq, bq)


def _edge_body(ea_ref, qi_ref, wt_ref, bt_ref, wk_ref, bk_ref, wv_ref,
               bv_ref, cem_ref, v_ref, e_ref, a_ref, r16_ref, cnt_ref):
    i = pl.program_id(0)
    tf = jnp.dot(ea_ref[...], wt_ref[...], preferred_element_type=jnp.float32)
    tf = jnp.maximum(tf + bt_ref[...], 0.0)
    k = jnp.dot(tf, wk_ref[...], preferred_element_type=jnp.float32) + bk_ref[...]
    v_ref[...] = jnp.dot(tf, wv_ref[...], preferred_element_type=jnp.float32) + bv_ref[...]
    sim = jnp.dot(tf, cem_ref[...], preferred_element_type=jnp.float32)
    best = sim[:, 0]
    bidx = jnp.zeros((EB,), jnp.int32)
    for c in range(1, NCL):
        m = sim[:, c] > best
        best = jnp.where(m, sim[:, c], best)
        bidx = jnp.where(m, jnp.int32(c), bidx)
    a_ref[...] = bidx
    att = jnp.sum(qi_ref[...] * k, axis=1) * SCAL
    ee = jnp.exp(att)
    e_ref[...] = ee
    col = lax.broadcasted_iota(jnp.int32, (EB, 16), 1)
    a2 = bidx[:, None]
    r16_ref[...] = (jnp.where(col == a2, ee[:, None], 0.0)
                    + jnp.where(col == a2 + NCL, 1.0, 0.0))
    onehot = (lax.broadcasted_iota(jnp.int32, (EB, NCL), 1) == a2)
    part = jnp.sum(onehot.astype(jnp.float32), axis=0, keepdims=True)

    @pl.when(i == 0)
    def _():
        cnt_ref[...] = jnp.zeros_like(cnt_ref)

    cnt_ref[...] += part


def _edge_call(ea, qi, wt, bt, wk, bk, wv, bv, cem):
    return pl.pallas_call(
        _edge_body,
        grid=(E // EB,),
        in_specs=[
            pl.BlockSpec((EB, 16), lambda i: (i, 0)),
            pl.BlockSpec((EB, AD), lambda i: (i, 0)),
            pl.BlockSpec((16, AD), lambda i: (0, 0)),
            pl.BlockSpec((1, AD), lambda i: (0, 0)),
            pl.BlockSpec((AD, AD), lambda i: (0, 0)),
            pl.BlockSpec((1, AD), lambda i: (0, 0)),
            pl.BlockSpec((AD, AD), lambda i: (0, 0)),
            pl.BlockSpec((1, AD), lambda i: (0, 0)),
            pl.BlockSpec((AD, NCL), lambda i: (0, 0)),
        ],
        out_specs=[
            pl.BlockSpec((EB, AD), lambda i: (i, 0)),
            pl.BlockSpec((EB,), lambda i: (i,)),
            pl.BlockSpec((EB,), lambda i: (i,)),
            pl.BlockSpec((EB, 16), lambda i: (i, 0)),
            pl.BlockSpec((1, NCL), lambda i: (0, 0)),
        ],
        out_shape=[
            jax.ShapeDtypeStruct((E, AD), jnp.float32),
            jax.ShapeDtypeStruct((E,), jnp.float32),
            jax.ShapeDtypeStruct((E,), jnp.int32),
            jax.ShapeDtypeStruct((E, 16), jnp.float32),
            jax.ShapeDtypeStruct((1, NCL), jnp.float32),
        ],
    )(ea, qi, wt, bt, wk, bk, wv, bv, cem)


_GQ_CH = 1000


@functools.partial(
    pl.kernel,
    out_type=jax.ShapeDtypeStruct((E, AD), jnp.float32),
    mesh=_mesh,
    compiler_params=pltpu.CompilerParams(use_tc_tiling_on_sc=False, needs_layout_passes=False),
    scratch_types=[
        pltpu.VMEM((_GQ_CH,), jnp.int32),
        pltpu.VMEM((_GQ_CH, AD), jnp.float32),
        pltpu.SemaphoreType.DMA,
    ],
)
def _gather_q(q_hbm, src_hbm, out_hbm, idx_v, rows_v, sem):
    wid = lax.axis_index("s") * 2 + lax.axis_index("c")
    base = wid * EPT

    def body(i, carry):
        off = pl.multiple_of(base + i * _GQ_CH, 8)
        pltpu.sync_copy(src_hbm.at[pl.ds(off, _GQ_CH)], idx_v)
        pltpu.async_copy(q_hbm.at[idx_v], rows_v, sem).wait()
        pltpu.sync_copy(rows_v, out_hbm.at[pl.ds(off, _GQ_CH)])
        return carry

    lax.fori_loop(0, EPT // _GQ_CH, body, 0)


_H_CH = 2000


@functools.partial(
    pl.kernel,
    out_type=jax.ShapeDtypeStruct((2, N, 16), jnp.float32),
    mesh=_mesh,
    compiler_params=pltpu.CompilerParams(use_tc_tiling_on_sc=False, needs_layout_passes=False),
    scratch_types=[
        pltpu.VMEM((_H_CH, 16), jnp.float32),
        pltpu.VMEM((_H_CH,), jnp.int32),
        pltpu.VMEM_SHARED((N, 16), jnp.float32),
    ],
)
def _hist(r16_hbm, src_hbm, z_hbm, out_hbm, rows_v, idx_v, bins_sh):
    cid = lax.axis_index("c")
    sid = lax.axis_index("s")
    wid = sid * 2 + cid

    @pl.when(sid == 0)
    def _():
        pltpu.sync_copy(z_hbm, bins_sh)

    plsc.subcore_barrier()

    def body(i, carry):
        off = pl.multiple_of(wid * EPT + i * _H_CH, 8)
        pltpu.sync_copy(src_hbm.at[pl.ds(off, _H_CH)], idx_v)
        pltpu.sync_copy(r16_hbm.at[pl.ds(off, _H_CH)], rows_v)
        pltpu.sync_copy(rows_v, bins_sh.at[idx_v], add=True)
        return carry

    lax.fori_loop(0, EPT // _H_CH, body, 0)
    plsc.subcore_barrier()
    rpt = N // 16
    ro = pl.multiple_of(sid * rpt, 8)
    pltpu.sync_copy(bins_sh.at[pl.ds(ro, rpt)], out_hbm.at[cid].at[pl.ds(ro, rpt)])


_F_CH = 400


def _splat(vec, j):
    idx = jnp.full((16, 1), j, jnp.int32)
    dn = lax.GatherDimensionNumbers(
        offset_dims=(), collapsed_slice_dims=(0,), start_index_map=(0,))
    return lax.gather(vec, idx, dn, (1,),
                      mode=lax.GatherScatterMode.PROMISE_IN_BOUNDS)


@functools.partial(
    pl.kernel,
    out_type=jax.ShapeDtypeStruct((2, N, AD), jnp.float32),
    mesh=_mesh,
    compiler_params=pltpu.CompilerParams(use_tc_tiling_on_sc=False, needs_layout_passes=False),
    scratch_types=[
        pltpu.VMEM((_F_CH, 16), jnp.float32),
        pltpu.VMEM((_F_CH, 16), jnp.float32),
        pltpu.VMEM((_F_CH,), jnp.int32),
        pltpu.VMEM((_F_CH,), jnp.int32),
        pltpu.VMEM((_F_CH,), jnp.float32),
        pltpu.VMEM((_F_CH, AD), jnp.float32),
        pltpu.VMEM_SHARED((N, AD), jnp.float32),
        pltpu.SemaphoreType.DMA,
    ],
)
def _combine(b0_hbm, b1_hbm, src_hbm, asg_hbm, e_hbm, v_hbm, z_hbm, out_hbm,
             r0_v, r1_v, src_v, asg_v, e_v, rows_v, comb_sh, sem):
    cid = lax.axis_index("c")
    sid = lax.axis_index("s")
    wid = sid * 2 + cid

    @pl.when(sid == 0)
    def _():
        pltpu.sync_copy(z_hbm, comb_sh)

    plsc.subcore_barrier()

    def body(i, carry):
        off = pl.multiple_of(wid * EPT + i * _F_CH, 8)
        pltpu.sync_copy(src_hbm.at[pl.ds(off, _F_CH)], src_v)
        pltpu.sync_copy(asg_hbm.at[pl.ds(off, _F_CH)], asg_v)
        pltpu.sync_copy(e_hbm.at[pl.ds(off, _F_CH)], e_v)
        pltpu.sync_copy(v_hbm.at[pl.ds(off, _F_CH)], rows_v)
        pltpu.async_copy(b0_hbm.at[src_v], r0_v, sem).wait()
        pltpu.async_copy(b1_hbm.at[src_v], r1_v, sem).wait()

        def grp(g, c2):
            gb = pl.multiple_of(g * 16, 8)
            rows = lax.broadcasted_iota(jnp.int32, (16,), 0) + gb
            a16 = asg_v[pl.ds(gb, 16)]
            den = (plsc.load_gather(r0_v, [rows, a16])
                   + plsc.load_gather(r1_v, [rows, a16]))
            cnt = (plsc.load_gather(r0_v, [rows, a16 + NCL])
                   + plsc.load_gather(r1_v, [rows, a16 + NCL]))
            den = jnp.where(den > 0.0, den, 1.0)
            cnt = jnp.maximum(cnt, 1.0)
            e16 = e_v[pl.ds(gb, 16)]
            coef = e16 / den / cnt
            for j in range(16):
                cf = _splat(coef, j)
                for d in range(AD // 16):
                    rows_v[gb + j, pl.ds(d * 16, 16)] = (
                        rows_v[gb + j, pl.ds(d * 16, 16)] * cf)
            return c2

        lax.fori_loop(0, _F_CH // 16, grp, 0)
        pltpu.sync_copy(rows_v, comb_sh.at[src_v], add=True)
        return carry

    lax.fori_loop(0, EPT // _F_CH, body, 0)
    plsc.subcore_barrier()
    rpt = N // 16
    ro = pl.multiple_of(sid * rpt, 8)
    pltpu.sync_copy(comb_sh.at[pl.ds(ro, rpt)], out_hbm.at[cid].at[pl.ds(ro, rpt)])


def _final_body(x_ref, c0_ref, c1_ref, cnt_ref, wo_ref, bo_ref, wu_ref,
                bu_ref, out_ref):
    ne = jnp.sum((cnt_ref[...] > 0.0).astype(jnp.float32))
    comb = (c0_ref[...] + c1_ref[...]) / jnp.maximum(ne, 1.0)
    fused = jnp.dot(comb, wo_ref[...], preferred_element_type=jnp.float32)
    fused = jnp.maximum(fused + bo_ref[...], 0.0)
    out = jnp.dot(fused, wu_ref[...], preferred_element_type=jnp.float32)
    out_ref[...] = x_ref[...] + out + bu_ref[...]


def _final_call(x, c0, c1, cnt8, wo, bo, wu, bu):
    return pl.pallas_call(
        _final_body,
        grid=(N // NB,),
        in_specs=[
            pl.BlockSpec((NB, IN_CH), lambda i: (i, 0)),
            pl.BlockSpec((NB, AD), lambda i: (i, 0)),
            pl.BlockSpec((NB, AD), lambda i: (i, 0)),
            pl.BlockSpec((1, NCL), lambda i: (0, 0)),
            pl.BlockSpec((AD, AD), lambda i: (0, 0)),
            pl.BlockSpec((1, AD), lambda i: (0, 0)),
            pl.BlockSpec((AD, IN_CH), lambda i: (0, 0)),
            pl.BlockSpec((1, IN_CH), lambda i: (0, 0)),
        ],
        out_specs=pl.BlockSpec((NB, IN_CH), lambda i: (i, 0)),
        out_shape=jax.ShapeDtypeStruct((N, IN_CH), jnp.float32),
    )(x, c0, c1, cnt8, wo, bo, wu, bu)


@jax.jit
def kernel(x, edge_index, edge_attr, W_down, b_down, W_up, b_up, W_time,
           b_time, W_q, b_q, W_k, b_k, W_v, b_v, cluster_emb, W_out, b_out):
    src = edge_index[0].astype(jnp.int32)
    q = _q_call(x, W_down, b_down.reshape(1, AD), W_q, b_q.reshape(1, AD))
    qi = _gather_q(q, src)
    v, e, assign, r16, cnt8 = _edge_call(
        edge_attr, qi, W_time, b_time.reshape(1, AD), W_k, b_k.reshape(1, AD),
        W_v, b_v.reshape(1, AD), cluster_emb.T)
    bins = _hist(r16, src, jnp.zeros((N, 16), jnp.float32))
    comb = _combine(bins[0], bins[1], src, assign, e, v,
                    jnp.zeros((N, AD), jnp.float32))
    return _final_call(x, comb[0], comb[1], cnt8, W_out,
                       b_out.reshape(1, AD), W_up, b_up.reshape(1, IN_CH))


# speedup vs baseline: 23.8438x; 1.0805x over previous
import functools

import jax
import jax.numpy as jnp
from jax import lax
from jax.experimental import pallas as pl
from jax.experimental.pallas import tpu as pltpu
from jax.experimental.pallas import tpu_sc as plsc

N = 10000
E = 320000
IN_CH = 128
AD = 64
NCL = 8
SCAL = float(AD) ** (-0.5)

NWORK = 32
EPT = E // NWORK
NB = 1000
EB = 2000
NEB = E // EB

_mesh = plsc.VectorSubcoreMesh(core_axis_name="c", subcore_axis_name="s")


def _q_body(x_ref, wd_ref, bd_ref, wq_ref, bq_ref, q_ref):
    nf = jnp.dot(x_ref[...], wd_ref[...], preferred_element_type=jnp.float32)
    nf = jnp.maximum(nf + bd_ref[...], 0.0)
    q_ref[...] = jnp.dot(nf, wq_ref[...], preferred_element_type=jnp.float32) + bq_ref[...]


def _q_call(x, wd, bd, wq, bq):
    return pl.pallas_call(
        _q_body,
        grid=(N // NB,),
        in_specs=[
            pl.BlockSpec((NB, IN_CH), lambda i: (i, 0)),
            pl.BlockSpec((IN_CH, AD), lambda i: (0, 0)),
            pl.BlockSpec((1, AD), lambda i: (0, 0)),
            pl.BlockSpec((AD, AD), lambda i: (0, 0)),
            pl.BlockSpec((1, AD), lambda i: (0, 0)),
        ],
        out_specs=pl.BlockSpec((NB, AD), lambda i: (i, 0)),
        out_shape=jax.ShapeDtypeStruct((N, AD), jnp.float32),
    )(x, wd, bd, wq, bq)


def _edge_body(ea_ref, qi_ref, wt_ref, bt_ref, wk_ref, bk_ref, wv_ref,
               bv_ref, cem_ref, v_ref, e_ref, a_ref, r16_ref, cnt_ref):
    i = pl.program_id(0)
    tf = jnp.dot(ea_ref[...], wt_ref[...], preferred_element_type=jnp.float32)
    tf = jnp.maximum(tf + bt_ref[...], 0.0)
    k = jnp.dot(tf, wk_ref[...], preferred_element_type=jnp.float32) + bk_ref[...]
    v_ref[...] = jnp.dot(tf, wv_ref[...], preferred_element_type=jnp.float32) + bv_ref[...]
    sim = jnp.dot(tf, cem_ref[...], preferred_element_type=jnp.float32)
    best = sim[:, 0]
    bidx = jnp.zeros((EB,), jnp.int32)
    for c in range(1, NCL):
        m = sim[:, c] > best
        best = jnp.where(m, sim[:, c], best)
        bidx = jnp.where(m, jnp.int32(c), bidx)
    a_ref[...] = bidx[None, None, :]
    att = jnp.sum(qi_ref[...] * k, axis=1) * SCAL
    ee = jnp.exp(att)
    e_ref[...] = ee[None, None, :]
    col = lax.broadcasted_iota(jnp.int32, (EB, 16), 1)
    a2 = bidx[:, None]
    r16_ref[...] = (jnp.where(col == a2, ee[:, None], 0.0)
                    + jnp.where(col == a2 + NCL, 1.0, 0.0))
    onehot = (lax.broadcasted_iota(jnp.int32, (EB, NCL), 1) == a2)
    part = jnp.sum(onehot.astype(jnp.float32), axis=0, keepdims=True)

    @pl.when(i == 0)
    def _():
        cnt_ref[...] = jnp.zeros_like(cnt_ref)

    cnt_ref[...] += part


def _edge_call(ea, qi, wt, bt, wk, bk, wv, bv, cem):
    return pl.pallas_call(
        _edge_body,
        grid=(E // EB,),
        in_specs=[
            pl.BlockSpec((EB, 16), lambda i: (i, 0)),
            pl.BlockSpec((EB, AD), lambda i: (i, 0)),
            pl.BlockSpec((16, AD), lambda i: (0, 0)),
            pl.BlockSpec((1, AD), lambda i: (0, 0)),
            pl.BlockSpec((AD, AD), lambda i: (0, 0)),
            pl.BlockSpec((1, AD), lambda i: (0, 0)),
            pl.BlockSpec((AD, AD), lambda i: (0, 0)),
            pl.BlockSpec((1, AD), lambda i: (0, 0)),
            pl.BlockSpec((AD, NCL), lambda i: (0, 0)),
        ],
        out_specs=[
            pl.BlockSpec((EB, AD), lambda i: (i, 0)),
            pl.BlockSpec((1, 1, EB), lambda i: (i, 0, 0)),
            pl.BlockSpec((1, 1, EB), lambda i: (i, 0, 0)),
            pl.BlockSpec((EB, 16), lambda i: (i, 0)),
            pl.BlockSpec((1, NCL), lambda i: (0, 0)),
        ],
        out_shape=[
            jax.ShapeDtypeStruct((E, AD), jnp.float32),
            jax.ShapeDtypeStruct((NEB, 1, EB), jnp.float32),
            jax.ShapeDtypeStruct((NEB, 1, EB), jnp.int32),
            jax.ShapeDtypeStruct((E, 16), jnp.float32),
            jax.ShapeDtypeStruct((1, NCL), jnp.float32),
        ],
    )(ea, qi, wt, bt, wk, bk, wv, bv, cem)


_GQ_CH = 1000


@functools.partial(
    pl.kernel,
    out_type=jax.ShapeDtypeStruct((E, AD), jnp.float32),
    mesh=_mesh,
    compiler_params=pltpu.CompilerParams(use_tc_tiling_on_sc=False, needs_layout_passes=False),
    scratch_types=[
        pltpu.VMEM((_GQ_CH,), jnp.int32),
        pltpu.VMEM((_GQ_CH, AD), jnp.float32),
        pltpu.SemaphoreType.DMA,
    ],
)
def _gather_q(q_hbm, src_hbm, out_hbm, idx_v, rows_v, sem):
    wid = lax.axis_index("s") * 2 + lax.axis_index("c")
    base = wid * EPT

    def body(i, carry):
        off = pl.multiple_of(base + i * _GQ_CH, 8)
        pltpu.sync_copy(src_hbm.at[pl.ds(off, _GQ_CH)], idx_v)
        pltpu.async_copy(q_hbm.at[idx_v], rows_v, sem).wait()
        pltpu.sync_copy(rows_v, out_hbm.at[pl.ds(off, _GQ_CH)])
        return carry

    lax.fori_loop(0, EPT // _GQ_CH, body, 0)


_H_CH = 2000


@functools.partial(
    pl.kernel,
    out_type=jax.ShapeDtypeStruct((2, N, 16), jnp.float32),
    mesh=_mesh,
    compiler_params=pltpu.CompilerParams(use_tc_tiling_on_sc=False, needs_layout_passes=False),
    scratch_types=[
        pltpu.VMEM((_H_CH, 16), jnp.float32),
        pltpu.VMEM((_H_CH,), jnp.int32),
        pltpu.VMEM_SHARED((N, 16), jnp.float32),
    ],
)
def _hist(r16_hbm, src_hbm, z_hbm, out_hbm, rows_v, idx_v, bins_sh):
    cid = lax.axis_index("c")
    sid = lax.axis_index("s")
    wid = sid * 2 + cid

    @pl.when(sid == 0)
    def _():
        pltpu.sync_copy(z_hbm, bins_sh)

    plsc.subcore_barrier()

    def body(i, carry):
        off = pl.multiple_of(wid * EPT + i * _H_CH, 8)
        pltpu.sync_copy(src_hbm.at[pl.ds(off, _H_CH)], idx_v)
        pltpu.sync_copy(r16_hbm.at[pl.ds(off, _H_CH)], rows_v)
        pltpu.sync_copy(rows_v, bins_sh.at[idx_v], add=True)
        return carry

    lax.fori_loop(0, EPT // _H_CH, body, 0)
    plsc.subcore_barrier()
    rpt = N // 16
    ro = pl.multiple_of(sid * rpt, 8)
    pltpu.sync_copy(bins_sh.at[pl.ds(ro, rpt)], out_hbm.at[cid].at[pl.ds(ro, rpt)])


_F_CH = 400


def _splat(vec, j):
    idx = jnp.full((16, 1), j, jnp.int32)
    dn = lax.GatherDimensionNumbers(
        offset_dims=(), collapsed_slice_dims=(0,), start_index_map=(0,))
    return lax.gather(vec, idx, dn, (1,),
                      mode=lax.GatherScatterMode.PROMISE_IN_BOUNDS)


@functools.partial(
    pl.kernel,
    out_type=jax.ShapeDtypeStruct((2, N, AD), jnp.float32),
    mesh=_mesh,
    compiler_params=pltpu.CompilerParams(use_tc_tiling_on_sc=False, needs_layout_passes=False),
    scratch_types=[
        pltpu.VMEM((_F_CH, 16), jnp.float32),
        pltpu.VMEM((_F_CH, 16), jnp.float32),
        pltpu.VMEM((_F_CH,), jnp.int32),
        pltpu.VMEM((_F_CH,), jnp.int32),
        pltpu.VMEM((_F_CH,), jnp.float32),
        pltpu.VMEM((_F_CH, AD), jnp.float32),
        pltpu.VMEM_SHARED((N, AD), jnp.float32),
        pltpu.SemaphoreType.DMA,
    ],
)
def _combine(b0_hbm, b1_hbm, src_hbm, asg_hbm, e_hbm, v_hbm, z_hbm, out_hbm,
             r0_v, r1_v, src_v, asg_v, e_v, rows_v, comb_sh, sem):
    cid = lax.axis_index("c")
    sid = lax.axis_index("s")
    wid = sid * 2 + cid

    @pl.when(sid == 0)
    def _():
        pltpu.sync_copy(z_hbm, comb_sh)

    plsc.subcore_barrier()

    def body(i, carry):
        off = pl.multiple_of(wid * EPT + i * _F_CH, 8)
        pltpu.sync_copy(src_hbm.at[pl.ds(off, _F_CH)], src_v)
        pltpu.sync_copy(asg_hbm.at[pl.ds(off, _F_CH)], asg_v)
        pltpu.sync_copy(e_hbm.at[pl.ds(off, _F_CH)], e_v)
        pltpu.sync_copy(v_hbm.at[pl.ds(off, _F_CH)], rows_v)
        pltpu.async_copy(b0_hbm.at[src_v], r0_v, sem).wait()
        pltpu.async_copy(b1_hbm.at[src_v], r1_v, sem).wait()

        def grp(g, c2):
            gb = pl.multiple_of(g * 16, 8)
            rows = lax.broadcasted_iota(jnp.int32, (16,), 0) + gb
            a16 = asg_v[pl.ds(gb, 16)]
            den = (plsc.load_gather(r0_v, [rows, a16])
                   + plsc.load_gather(r1_v, [rows, a16]))
            cnt = (plsc.load_gather(r0_v, [rows, a16 + NCL])
                   + plsc.load_gather(r1_v, [rows, a16 + NCL]))
            den = jnp.where(den > 0.0, den, 1.0)
            cnt = jnp.maximum(cnt, 1.0)
            e16 = e_v[pl.ds(gb, 16)]
            coef = e16 / den / cnt
            for j in range(16):
                cf = _splat(coef, j)
                for d in range(AD // 16):
                    rows_v[gb + j, pl.ds(d * 16, 16)] = (
                        rows_v[gb + j, pl.ds(d * 16, 16)] * cf)
            return c2

        lax.fori_loop(0, _F_CH // 16, grp, 0)
        pltpu.sync_copy(rows_v, comb_sh.at[src_v], add=True)
        return carry

    lax.fori_loop(0, EPT // _F_CH, body, 0)
    plsc.subcore_barrier()
    rpt = N // 16
    ro = pl.multiple_of(sid * rpt, 8)
    pltpu.sync_copy(comb_sh.at[pl.ds(ro, rpt)], out_hbm.at[cid].at[pl.ds(ro, rpt)])


def _final_body(x_ref, c0_ref, c1_ref, cnt_ref, wo_ref, bo_ref, wu_ref,
                bu_ref, out_ref):
    ne = jnp.sum((cnt_ref[...] > 0.0).astype(jnp.float32))
    comb = (c0_ref[...] + c1_ref[...]) / jnp.maximum(ne, 1.0)
    fused = jnp.dot(comb, wo_ref[...], preferred_element_type=jnp.float32)
    fused = jnp.maximum(fused + bo_ref[...], 0.0)
    out = jnp.dot(fused, wu_ref[...], preferred_element_type=jnp.float32)
    out_ref[...] = x_ref[...] + out + bu_ref[...]


def _final_call(x, c0, c1, cnt8, wo, bo, wu, bu):
    return pl.pallas_call(
        _final_body,
        grid=(N // NB,),
        in_specs=[
            pl.BlockSpec((NB, IN_CH), lambda i: (i, 0)),
            pl.BlockSpec((NB, AD), lambda i: (i, 0)),
            pl.BlockSpec((NB, AD), lambda i: (i, 0)),
            pl.BlockSpec((1, NCL), lambda i: (0, 0)),
            pl.BlockSpec((AD, AD), lambda i: (0, 0)),
            pl.BlockSpec((1, AD), lambda i: (0, 0)),
            pl.BlockSpec((AD, IN_CH), lambda i: (0, 0)),
            pl.BlockSpec((1, IN_CH), lambda i: (0, 0)),
        ],
        out_specs=pl.BlockSpec((NB, IN_CH), lambda i: (i, 0)),
        out_shape=jax.ShapeDtypeStruct((N, IN_CH), jnp.float32),
    )(x, c0, c1, cnt8, wo, bo, wu, bu)


@jax.jit
def kernel(x, edge_index, edge_attr, W_down, b_down, W_up, b_up, W_time,
           b_time, W_q, b_q, W_k, b_k, W_v, b_v, cluster_emb, W_out, b_out):
    src = edge_index[0].astype(jnp.int32)
    q = _q_call(x, W_down, b_down.reshape(1, AD), W_q, b_q.reshape(1, AD))
    qi = _gather_q(q, src)
    v, e3, a3, r16, cnt8 = _edge_call(
        edge_attr, qi, W_time, b_time.reshape(1, AD), W_k, b_k.reshape(1, AD),
        W_v, b_v.reshape(1, AD), cluster_emb.T)
    e = e3.reshape(E)
    assign = a3.reshape(E)
    bins = _hist(r16, src, jnp.zeros((N, 16), jnp.float32))
    comb = _combine(bins[0], bins[1], src, assign, e, v,
                    jnp.zeros((N, AD), jnp.float32))
    return _final_call(x, comb[0], comb[1], cnt8, W_out,
                       b_out.reshape(1, AD), W_up, b_up.reshape(1, IN_CH))


# speedup vs baseline: 24.4875x; 1.0270x over previous
import functools

import jax
import jax.numpy as jnp
from jax import lax
from jax.experimental import pallas as pl
from jax.experimental.pallas import tpu as pltpu
from jax.experimental.pallas import tpu_sc as plsc

N = 10000
E = 320000
IN_CH = 128
AD = 64
NCL = 8
SCAL = float(AD) ** (-0.5)

NWORK = 32
EPT = E // NWORK
NB = 1000
EB = 2000
NEB = E // EB

_mesh = plsc.VectorSubcoreMesh(core_axis_name="c", subcore_axis_name="s")


def _q_body(x_ref, wd_ref, bd_ref, wq_ref, bq_ref, q_ref):
    nf = jnp.dot(x_ref[...], wd_ref[...], preferred_element_type=jnp.float32)
    nf = jnp.maximum(nf + bd_ref[...], 0.0)
    q_ref[...] = jnp.dot(nf, wq_ref[...], preferred_element_type=jnp.float32) + bq_ref[...]


def _q_call(x, wd, bd, wq, bq):
    return pl.pallas_call(
        _q_body,
        grid=(N // NB,),
        in_specs=[
            pl.BlockSpec((NB, IN_CH), lambda i: (i, 0)),
            pl.BlockSpec((IN_CH, AD), lambda i: (0, 0)),
            pl.BlockSpec((1, AD), lambda i: (0, 0)),
            pl.BlockSpec((AD, AD), lambda i: (0, 0)),
            pl.BlockSpec((1, AD), lambda i: (0, 0)),
        ],
        out_specs=pl.BlockSpec((NB, AD), lambda i: (i, 0)),
        out_shape=jax.ShapeDtypeStruct((N, AD), jnp.float32),
    )(x, wd, bd, wq, bq)


def _edge_body(ea_ref, qi_ref, wt_ref, bt_ref, wk_ref, bk_ref, wv_ref,
               bv_ref, cem_ref, v_ref, e_ref, a_ref, r16_ref, cnt_ref):
    i = pl.program_id(0)
    tf = jnp.dot(ea_ref[...], wt_ref[...], preferred_element_type=jnp.float32)
    tf = jnp.maximum(tf + bt_ref[...], 0.0)
    k = jnp.dot(tf, wk_ref[...], preferred_element_type=jnp.float32) + bk_ref[...]
    v_ref[...] = jnp.dot(tf, wv_ref[...], preferred_element_type=jnp.float32) + bv_ref[...]
    sim = jnp.dot(tf, cem_ref[...], preferred_element_type=jnp.float32)
    best = sim[:, 0]
    bidx = jnp.zeros((EB,), jnp.int32)
    for c in range(1, NCL):
        m = sim[:, c] > best
        best = jnp.where(m, sim[:, c], best)
        bidx = jnp.where(m, jnp.int32(c), bidx)
    a_ref[...] = bidx[None, None, :]
    att = jnp.sum(qi_ref[...] * k, axis=1) * SCAL
    ee = jnp.exp(att)
    e_ref[...] = ee[None, None, :]
    col = lax.broadcasted_iota(jnp.int32, (EB, 16), 1)
    a2 = bidx[:, None]
    r16_ref[...] = (jnp.where(col == a2, ee[:, None], 0.0)
                    + jnp.where(col == a2 + NCL, 1.0, 0.0))
    onehot = (lax.broadcasted_iota(jnp.int32, (EB, NCL), 1) == a2)
    part = jnp.sum(onehot.astype(jnp.float32), axis=0, keepdims=True)

    @pl.when(i == 0)
    def _():
        cnt_ref[...] = jnp.zeros_like(cnt_ref)

    cnt_ref[...] += part


def _edge_call(ea, qi, wt, bt, wk, bk, wv, bv, cem):
    return pl.pallas_call(
        _edge_body,
        grid=(E // EB,),
        in_specs=[
            pl.BlockSpec((EB, 16), lambda i: (i, 0)),
            pl.BlockSpec((EB, AD), lambda i: (i, 0)),
            pl.BlockSpec((16, AD), lambda i: (0, 0)),
            pl.BlockSpec((1, AD), lambda i: (0, 0)),
            pl.BlockSpec((AD, AD), lambda i: (0, 0)),
            pl.BlockSpec((1, AD), lambda i: (0, 0)),
            pl.BlockSpec((AD, AD), lambda i: (0, 0)),
            pl.BlockSpec((1, AD), lambda i: (0, 0)),
            pl.BlockSpec((AD, NCL), lambda i: (0, 0)),
        ],
        out_specs=[
            pl.BlockSpec((EB, AD), lambda i: (i, 0)),
            pl.BlockSpec((1, 1, EB), lambda i: (i, 0, 0)),
            pl.BlockSpec((1, 1, EB), lambda i: (i, 0, 0)),
            pl.BlockSpec((EB, 16), lambda i: (i, 0)),
            pl.BlockSpec((1, NCL), lambda i: (0, 0)),
        ],
        out_shape=[
            jax.ShapeDtypeStruct((E, AD), jnp.float32),
            jax.ShapeDtypeStruct((NEB, 1, EB), jnp.float32),
            jax.ShapeDtypeStruct((NEB, 1, EB), jnp.int32),
            jax.ShapeDtypeStruct((E, 16), jnp.float32),
            jax.ShapeDtypeStruct((1, NCL), jnp.float32),
        ],
    )(ea, qi, wt, bt, wk, bk, wv, bv, cem)


_GQ_CH = 1000


@functools.partial(
    pl.kernel,
    out_type=jax.ShapeDtypeStruct((E, AD), jnp.float32),
    mesh=_mesh,
    compiler_params=pltpu.CompilerParams(use_tc_tiling_on_sc=False, needs_layout_passes=False),
    scratch_types=[
        pltpu.VMEM((_GQ_CH,), jnp.int32),
        pltpu.VMEM((_GQ_CH, AD), jnp.float32),
        pltpu.SemaphoreType.DMA,
    ],
)
def _gather_q(q_hbm, src_hbm, out_hbm, idx_v, rows_v, sem):
    wid = lax.axis_index("s") * 2 + lax.axis_index("c")
    base = wid * EPT

    def body(i, carry):
        off = pl.multiple_of(base + i * _GQ_CH, 8)
        pltpu.sync_copy(src_hbm.at[pl.ds(off, _GQ_CH)], idx_v)
        pltpu.async_copy(q_hbm.at[idx_v], rows_v, sem).wait()
        pltpu.sync_copy(rows_v, out_hbm.at[pl.ds(off, _GQ_CH)])
        return carry

    lax.fori_loop(0, EPT // _GQ_CH, body, 0)


_H_CH = 2000


@functools.partial(
    pl.kernel,
    out_type=jax.ShapeDtypeStruct((2, N, 16), jnp.float32),
    mesh=_mesh,
    compiler_params=pltpu.CompilerParams(use_tc_tiling_on_sc=False, needs_layout_passes=False),
    scratch_types=[
        pltpu.VMEM((_H_CH, 16), jnp.float32),
        pltpu.VMEM((_H_CH,), jnp.int32),
        pltpu.VMEM_SHARED((N, 16), jnp.float32),
        pltpu.SemaphoreType.DMA,
    ],
)
def _hist(r16_hbm, src_hbm, z_hbm, out_hbm, rows_v, idx_v, bins_sh, sem):
    cid = lax.axis_index("c")
    sid = lax.axis_index("s")
    wid = sid * 2 + cid

    @pl.when(sid == 0)
    def _():
        pltpu.sync_copy(z_hbm, bins_sh)

    plsc.subcore_barrier()

    def body(i, carry):
        off = pl.multiple_of(wid * EPT + i * _H_CH, 8)
        d1 = pltpu.async_copy(src_hbm.at[pl.ds(off, _H_CH)], idx_v, sem)
        d2 = pltpu.async_copy(r16_hbm.at[pl.ds(off, _H_CH)], rows_v, sem)
        d1.wait()
        d2.wait()
        pltpu.sync_copy(rows_v, bins_sh.at[idx_v], add=True)
        return carry

    lax.fori_loop(0, EPT // _H_CH, body, 0)
    plsc.subcore_barrier()
    rpt = N // 16
    ro = pl.multiple_of(sid * rpt, 8)
    pltpu.sync_copy(bins_sh.at[pl.ds(ro, rpt)], out_hbm.at[cid].at[pl.ds(ro, rpt)])


_F_CH = 400


def _splat(vec, j):
    idx = jnp.full((16, 1), j, jnp.int32)
    dn = lax.GatherDimensionNumbers(
        offset_dims=(), collapsed_slice_dims=(0,), start_index_map=(0,))
    return lax.gather(vec, idx, dn, (1,),
                      mode=lax.GatherScatterMode.PROMISE_IN_BOUNDS)


@functools.partial(
    pl.kernel,
    out_type=jax.ShapeDtypeStruct((2, N, AD), jnp.float32),
    mesh=_mesh,
    compiler_params=pltpu.CompilerParams(use_tc_tiling_on_sc=False, needs_layout_passes=False),
    scratch_types=[
        pltpu.VMEM((_F_CH, 16), jnp.float32),
        pltpu.VMEM((_F_CH, 16), jnp.float32),
        pltpu.VMEM((_F_CH,), jnp.int32),
        pltpu.VMEM((_F_CH,), jnp.int32),
        pltpu.VMEM((_F_CH,), jnp.float32),
        pltpu.VMEM((_F_CH, AD), jnp.float32),
        pltpu.VMEM_SHARED((N, AD), jnp.float32),
        pltpu.SemaphoreType.DMA,
    ],
)
def _combine(b0_hbm, b1_hbm, src_hbm, asg_hbm, e_hbm, v_hbm, z_hbm, out_hbm,
             r0_v, r1_v, src_v, asg_v, e_v, rows_v, comb_sh, sem):
    cid = lax.axis_index("c")
    sid = lax.axis_index("s")
    wid = sid * 2 + cid

    @pl.when(sid == 0)
    def _():
        pltpu.sync_copy(z_hbm, comb_sh)

    plsc.subcore_barrier()

    def body(i, carry):
        off = pl.multiple_of(wid * EPT + i * _F_CH, 8)
        d1 = pltpu.async_copy(src_hbm.at[pl.ds(off, _F_CH)], src_v, sem)
        d2 = pltpu.async_copy(asg_hbm.at[pl.ds(off, _F_CH)], asg_v, sem)
        d3 = pltpu.async_copy(e_hbm.at[pl.ds(off, _F_CH)], e_v, sem)
        d4 = pltpu.async_copy(v_hbm.at[pl.ds(off, _F_CH)], rows_v, sem)
        d1.wait()
        d2.wait()
        d3.wait()
        d4.wait()
        g1 = pltpu.async_copy(b0_hbm.at[src_v], r0_v, sem)
        g2 = pltpu.async_copy(b1_hbm.at[src_v], r1_v, sem)
        g1.wait()
        g2.wait()

        def grp(g, c2):
            gb = pl.multiple_of(g * 16, 8)
            rows = lax.broadcasted_iota(jnp.int32, (16,), 0) + gb
            a16 = asg_v[pl.ds(gb, 16)]
            den = (plsc.load_gather(r0_v, [rows, a16])
                   + plsc.load_gather(r1_v, [rows, a16]))
            cnt = (plsc.load_gather(r0_v, [rows, a16 + NCL])
                   + plsc.load_gather(r1_v, [rows, a16 + NCL]))
            den = jnp.where(den > 0.0, den, 1.0)
            cnt = jnp.maximum(cnt, 1.0)
            e16 = e_v[pl.ds(gb, 16)]
            coef = e16 / den / cnt
            for j in range(16):
                cf = _splat(coef, j)
                for d in range(AD // 16):
                    rows_v[gb + j, pl.ds(d * 16, 16)] = (
                        rows_v[gb + j, pl.ds(d * 16, 16)] * cf)
            return c2

        lax.fori_loop(0, _F_CH // 16, grp, 0)
        pltpu.sync_copy(rows_v, comb_sh.at[src_v], add=True)
        return carry

    lax.fori_loop(0, EPT // _F_CH, body, 0)
    plsc.subcore_barrier()
    rpt = N // 16
    ro = pl.multiple_of(sid * rpt, 8)
    pltpu.sync_copy(comb_sh.at[pl.ds(ro, rpt)], out_hbm.at[cid].at[pl.ds(ro, rpt)])


def _final_body(x_ref, c0_ref, c1_ref, cnt_ref, wo_ref, bo_ref, wu_ref,
                bu_ref, out_ref):
    ne = jnp.sum((cnt_ref[...] > 0.0).astype(jnp.float32))
    comb = (c0_ref[...] + c1_ref[...]) / jnp.maximum(ne, 1.0)
    fused = jnp.dot(comb, wo_ref[...], preferred_element_type=jnp.float32)
    fused = jnp.maximum(fused + bo_ref[...], 0.0)
    out = jnp.dot(fused, wu_ref[...], preferred_element_type=jnp.float32)
    out_ref[...] = x_ref[...] + out + bu_ref[...]


def _final_call(x, c0, c1, cnt8, wo, bo, wu, bu):
    return pl.pallas_call(
        _final_body,
        grid=(N // NB,),
        in_specs=[
            pl.BlockSpec((NB, IN_CH), lambda i: (i, 0)),
            pl.BlockSpec((NB, AD), lambda i: (i, 0)),
            pl.BlockSpec((NB, AD), lambda i: (i, 0)),
            pl.BlockSpec((1, NCL), lambda i: (0, 0)),
            pl.BlockSpec((AD, AD), lambda i: (0, 0)),
            pl.BlockSpec((1, AD), lambda i: (0, 0)),
            pl.BlockSpec((AD, IN_CH), lambda i: (0, 0)),
            pl.BlockSpec((1, IN_CH), lambda i: (0, 0)),
        ],
        out_specs=pl.BlockSpec((NB, IN_CH), lambda i: (i, 0)),
        out_shape=jax.ShapeDtypeStruct((N, IN_CH), jnp.float32),
    )(x, c0, c1, cnt8, wo, bo, wu, bu)


@jax.jit
def kernel(x, edge_index, edge_attr, W_down, b_down, W_up, b_up, W_time,
           b_time, W_q, b_q, W_k, b_k, W_v, b_v, cluster_emb, W_out, b_out):
    src = edge_index[0].astype(jnp.int32)
    q = _q_call(x, W_down, b_down.reshape(1, AD), W_q, b_q.reshape(1, AD))
    qi = _gather_q(q, src)
    v, e3, a3, r16, cnt8 = _edge_call(
        edge_attr, qi, W_time, b_time.reshape(1, AD), W_k, b_k.reshape(1, AD),
        W_v, b_v.reshape(1, AD), cluster_emb.T)
    e = e3.reshape(E)
    assign = a3.reshape(E)
    bins = _hist(r16, src, jnp.zeros((N, 16), jnp.float32))
    comb = _combine(bins[0], bins[1], src, assign, e, v,
                    jnp.zeros((N, AD), jnp.float32))
    return _final_call(x, comb[0], comb[1], cnt8, W_out,
                       b_out.reshape(1, AD), W_up, b_up.reshape(1, IN_CH))
